# baseline (speedup 1.0000x reference)
import functools

import jax
import jax.numpy as jnp
from jax import lax
from jax.experimental import pallas as pl
from jax.experimental.pallas import tpu as pltpu
from jax.experimental.pallas import tpu_sc as plsc

N = 320000
F = 128
P = 10000
PP = 10240


BN = 512
GRID = N // BN


def _matvec_body(x_ref, w_ref, b_ref, r_ref, s_ref):
    x = x_ref[...]
    w = w_ref[...]
    r_ref[...] = jnp.sum(x * w, axis=1) + b_ref[0, 0]
    s_ref[...] = x[:, 0].astype(jnp.int32)


def _tc_matvec(i, w_row, b2):
    return pl.pallas_call(
        _matvec_body,
        grid=(GRID,),
        in_specs=[
            pl.BlockSpec((BN, 129), lambda g: (g, 0)),
            pl.BlockSpec((1, 129), lambda g: (0, 0)),
            pl.BlockSpec((1, 1), lambda g: (0, 0)),
        ],
        out_specs=[
            pl.BlockSpec((BN,), lambda g: (g,)),
            pl.BlockSpec((BN,), lambda g: (g,)),
        ],
        out_shape=[
            jax.ShapeDtypeStruct((N,), jnp.float32),
            jax.ShapeDtypeStruct((N,), jnp.int32),
        ],
    )(i, w_row, b2)


NT = 16
CHUNK = N // NT
TPP = PP // NT
L = 16


def _g16(x, idx):
    return jnp.take_along_axis(x, idx, axis=0, mode="promise_in_bounds")


def _segmax_body(r_hbm, s_hbm, out_hbm, r_v, s_v, c_loc, shared, mbuf, res_v):
    sid = lax.axis_index("s")
    base = sid * CHUNK
    pltpu.sync_copy(r_hbm.at[pl.ds(base, CHUNK)], r_v)
    pltpu.sync_copy(s_hbm.at[pl.ds(base, CHUNK)], s_v)

    zeros16 = jnp.zeros((L,), jnp.float32)

    def zero_body(k, c):
        c_loc[pl.ds(k * L, L)] = zeros16
        return c

    lax.fori_loop(0, PP // L, zero_body, 0)

    iota = lax.broadcasted_iota(jnp.int32, (L,), 0)
    nxt = jnp.minimum(iota + 1, L - 1)
    shift_idx = [jnp.maximum(iota - d, 0) for d in (1, 2, 4, 8)]
    last = iota == (L - 1)

    def body(k, c):
        off = k * L
        v = jnp.maximum(r_v[pl.ds(off, L)], 0.0)
        s = s_v[pl.ds(off, L)]
        for idx in shift_idx:
            sv = _g16(v, idx)
            ss = _g16(s, idx)
            v = jnp.where(ss == s, jnp.maximum(v, sv), v)
        m = (s != _g16(s, nxt)) | last
        old = plsc.load_gather(c_loc, [s], mask=m)
        plsc.store_scatter(c_loc, [s], jnp.maximum(old, v), mask=m)
        return c

    lax.fori_loop(0, CHUNK // L, body, 0)

    pltpu.sync_copy(c_loc, shared.at[sid])
    plsc.subcore_barrier()
    pltpu.sync_copy(shared.at[:, pl.ds(sid * TPP, TPP)], mbuf)

    def merge_body(j, c):
        acc = mbuf[0, pl.ds(j * L, L)]
        for t in range(1, NT):
            acc = jnp.maximum(acc, mbuf[t, pl.ds(j * L, L)])
        res_v[pl.ds(j * L, L)] = acc
        return c

    lax.fori_loop(0, TPP // L, merge_body, 0)
    pltpu.sync_copy(res_v, out_hbm.at[pl.ds(sid * TPP, TPP)])


_sc_segmax = functools.partial(
    pl.kernel,
    out_type=jax.ShapeDtypeStruct((PP,), jnp.float32),
    mesh=plsc.VectorSubcoreMesh(
        core_axis_name="c", subcore_axis_name="s", num_cores=1
    ),
    compiler_params=pltpu.CompilerParams(needs_layout_passes=False),
    scratch_types=[
        pltpu.VMEM((CHUNK,), jnp.float32),
        pltpu.VMEM((CHUNK,), jnp.int32),
        pltpu.VMEM((PP,), jnp.float32),
        pltpu.VMEM_SHARED((NT, PP), jnp.float32),
        pltpu.VMEM((NT, TPP), jnp.float32),
        pltpu.VMEM((TPP,), jnp.float32),
    ],
)(_segmax_body)


def kernel(_, i, p, W, b):
    w_row = jnp.concatenate([jnp.zeros((1, 1), jnp.float32), W], axis=0)
    w_row = w_row.reshape(1, F + 1)
    b2 = b.reshape(1, 1)
    r, seg = _tc_matvec(i, w_row, b2)
    c = _sc_segmax(r, seg)[:P]
    return jnp.where(jnp.arange(P) < p, c, jnp.zeros((), jnp.float32))
